# baseline (speedup 1.0000x reference)
import jax, jax.numpy as jnp
import numpy as np
from jax.experimental import pallas as pl
from jax.experimental.pallas import tpu as pltpu

_interpret = False

_EBLK = 1000


def _edge_dense_body(ss_ref, sd_ref, xe_ref,
                     We_ref, Wgv_ref, Wgd_ref,
                     e_ref, ge_ref, gd_ref):
    cat = jnp.concatenate([ss_ref[...], sd_ref[...], xe_ref[...]], axis=-1)
    e = jax.nn.gelu(jnp.dot(cat, We_ref[...],
                            preferred_element_type=jnp.float32))
    e_ref[...] = e
    ge_ref[...] = jnp.dot(e, Wgv_ref[...], preferred_element_type=jnp.float32)
    gd_ref[...] = jnp.dot(e, Wgd_ref[...], preferred_element_type=jnp.float32)


def _edge_dense(s_src, s_dst, xe, We, Wgv, Wgd):
    E = s_src.shape[0]
    grid = (E // _EBLK,)
    bs64 = pl.BlockSpec((_EBLK, 64), lambda i: (i, 0))
    bw = lambda r, c: pl.BlockSpec((r, c), lambda i: (0, 0))
    e, ge, gd = pl.pallas_call(
        _edge_dense_body,
        grid=grid,
        in_specs=[bs64, bs64, bs64,
                  bw(192, 64), bw(64, 64), bw(64, 64)],
        out_specs=[bs64, bs64, bs64],
        out_shape=[jax.ShapeDtypeStruct((E, 64), jnp.float32)] * 3,
        interpret=_interpret,
    )(s_src, s_dst, xe, We, Wgv, Wgd)
    return e, ge, gd


def _frames(x1, x2, x3):
    eps = 1e-6
    v1 = x3 - x2
    v2 = x1 - x2
    e1 = v1 / (jnp.linalg.norm(v1, axis=-1, keepdims=True) + eps)
    u2 = v2 - e1 * jnp.sum(e1 * v2, axis=-1, keepdims=True)
    e2 = u2 / (jnp.linalg.norm(u2, axis=-1, keepdims=True) + eps)
    e3 = jnp.cross(e1, e2)
    return jnp.stack([e1, e2, e3], axis=-1), x2


def kernel(timestep_encoding, noised_atm_coords, lig_atom_attr, lig_trp_attr,
           lig_af_pair_attr, lig_atom_pair_attr, gather_idx_i_molid,
           gather_idx_ijk_i, gather_idx_ijk_j, gather_idx_ijk_k,
           gather_idx_I_ijk, gather_idx_UI_I, gather_idx_UI_u,
           gather_idx_uv_u, gather_idx_uv_v, num_i, num_I, fourier_B,
           atm_embed_W1, atm_embed_b1, atm_embed_W2, atm_embed_b2,
           trp_embed_W, trp_vembed_W, blk_We, blk_Wgv, blk_Wgd, blk_Wus,
           blk_Wuv, out_atm_W, out_scale_W, out_scale_b):
    N = noised_atm_coords.shape[0]
    M = lig_trp_attr.shape[0]
    ts = timestep_encoding[gather_idx_i_molid]
    ang = 2.0 * np.pi * (ts @ fourier_B)
    tenc = jnp.concatenate([jnp.cos(ang), jnp.sin(ang)], axis=-1)
    h = jnp.concatenate([lig_atom_attr, tenc], axis=-1)
    h = jax.nn.gelu(h @ atm_embed_W1 + atm_embed_b1) @ atm_embed_W2 + atm_embed_b2
    atm_rep = jnp.concatenate([h[:, None, :], jnp.zeros((N, 3, h.shape[-1]), dtype=h.dtype)], axis=1)
    ti = gather_idx_ijk_i[gather_idx_I_ijk]
    tj = gather_idx_ijk_j[gather_idx_I_ijk]
    tk = gather_idx_ijk_k[gather_idx_I_ijk]
    coords = noised_atm_coords
    coords = coords + jnp.asarray(num_i - N + num_I - M).astype(coords.dtype)
    Rt, tt = _frames(coords[ti], coords[tj], coords[tk])
    trp_s = lig_trp_attr @ trp_embed_W
    trp_v = (lig_trp_attr @ trp_vembed_W).reshape(M, 3, -1)
    trp_rep = jnp.concatenate([trp_s[:, None, :], jnp.einsum('nij,njd->nid', Rt, trp_v)], axis=1)
    x = jnp.concatenate([atm_rep, trp_rep], axis=0)
    src = jnp.concatenate([gather_idx_UI_I + N, gather_idx_UI_u, gather_idx_uv_u])
    dst = jnp.concatenate([gather_idx_UI_u, gather_idx_UI_I + N, gather_idx_uv_v])
    x_edge = jnp.concatenate([lig_af_pair_attr, lig_af_pair_attr, lig_atom_pair_attr], axis=0)
    Nt = N + M
    n_stacks = blk_We.shape[0]
    eye = jnp.broadcast_to(jnp.eye(3, dtype=coords.dtype), (N, 3, 3))
    for l in range(n_stacks):
        Rt, tt = _frames(coords[ti], coords[tj], coords[tk])
        R_all = jnp.concatenate([eye, Rt], axis=0)
        t_all = jnp.concatenate([coords, tt], axis=0)
        s = x[:, 0]
        v = x[:, 1:]
        e, ge, gdd = _edge_dense(s[src], s[dst], x_edge,
                                 blk_We[l], blk_Wgv[l], blk_Wgd[l])
        agg_s = jax.ops.segment_sum(e, dst, num_segments=Nt)
        rel = t_all[src] - t_all[dst]
        rel_loc = jnp.einsum('eji,ej->ei', R_all[dst], rel)
        vmsg = v[src] * ge[:, None, :] + rel_loc[:, :, None] * gdd[:, None, :]
        agg_v = jax.ops.segment_sum(vmsg, dst, num_segments=Nt)
        s = s + jax.nn.gelu(jnp.concatenate([s, agg_s], axis=-1) @ blk_Wus[l])
        v = v + jnp.einsum('nid,de->nie', agg_v, blk_Wuv[l])
        x = jnp.concatenate([s[:, None, :], v], axis=1)
        atm = x[:N]
        scale = jax.nn.sigmoid(atm[:, 0] @ out_scale_W + out_scale_b) * 10.0
        ddir = (atm[:, 1:] @ out_atm_W)[..., 0]
        coords = coords + ddir * scale
    return coords


# speedup vs baseline: 1.0401x; 1.0401x over previous
import functools
import jax, jax.numpy as jnp
import numpy as np
from jax import lax
from jax.experimental import pallas as pl
from jax.experimental.pallas import tpu as pltpu
from jax.experimental.pallas import tpu_sc as plsc

_interpret = False

_NW = 32
_KG = 128


def _pad_rows(x, B):
    return jnp.pad(x, ((0, B - x.shape[0]),) + ((0, 0),) * (x.ndim - 1))


def _sc_gather(table, idx, G):
    B = idx.shape[0]
    D = table.shape[1]
    nb = B // _NW
    mesh = plsc.VectorSubcoreMesh(core_axis_name="c", subcore_axis_name="s")

    @functools.partial(
        pl.kernel, mesh=mesh,
        out_type=jax.ShapeDtypeStruct((B, D), jnp.float32),
        scratch_types=[
            pltpu.VMEM((_KG,), jnp.int32),
            pltpu.VMEM((_KG, D), jnp.float32),
            pltpu.SemaphoreType.DMA,
        ],
    )
    def k(table_hbm, idx_hbm, out_hbm, idx_v, rows_v, sem):
        wid = lax.axis_index("s") * 2 + lax.axis_index("c")
        base = wid * nb

        def body(g, carry):
            off = base + g * _KG
            pltpu.sync_copy(idx_hbm.at[pl.ds(off, _KG)], idx_v)
            pltpu.async_copy(table_hbm.at[idx_v], rows_v, sem).wait()
            pltpu.sync_copy(rows_v, out_hbm.at[pl.ds(off, _KG)])
            return carry

        lax.fori_loop(0, G, body, 0)

    return k(table, idx)


def _sc_scatter_add(data, idx2, zeros, G):
    B = data.shape[0]
    Hp = zeros.shape[0]
    nt = B // 16
    mesh = plsc.VectorSubcoreMesh(core_axis_name="c", subcore_axis_name="s")

    @functools.partial(
        pl.kernel, mesh=mesh,
        out_type=jax.ShapeDtypeStruct((2, Hp, 64), jnp.float32),
        scratch_types=[
            pltpu.VMEM((_KG,), jnp.int32),
            pltpu.VMEM((_KG, 64), jnp.float32),
            pltpu.VMEM_SHARED((Hp, 64), jnp.float32),
            pltpu.SemaphoreType.DMA,
        ],
    )
    def k(data_hbm, idx_hbm, zeros_hbm, out_hbm, idx_v, rows_v, acc, sem):
        c = lax.axis_index("c")
        sid = lax.axis_index("s")

        @pl.when(sid == 0)
        def _zero():
            pltpu.sync_copy(zeros_hbm, acc)

        plsc.subcore_barrier()
        base = sid * nt

        def body(g, carry):
            off = base + g * _KG
            pltpu.sync_copy(idx_hbm.at[c, pl.ds(off, _KG)], idx_v)
            pltpu.sync_copy(data_hbm.at[pl.ds(off, _KG)], rows_v)
            pltpu.sync_copy(rows_v, acc.at[idx_v], add=True)
            return carry

        lax.fori_loop(0, nt // _KG, body, 0)
        plsc.subcore_barrier()

        @pl.when(sid == 0)
        def _dump():
            pltpu.sync_copy(acc, out_hbm.at[c])

    return k(data, idx2, zeros)

_EBLK = 1000


def _edge_dense_body(ss_ref, sd_ref, xe_ref,
                     We_ref, Wgv_ref, Wgd_ref,
                     e_ref, ge_ref, gd_ref):
    cat = jnp.concatenate([ss_ref[...], sd_ref[...], xe_ref[...]], axis=-1)
    e = jax.nn.gelu(jnp.dot(cat, We_ref[...],
                            preferred_element_type=jnp.float32))
    e_ref[...] = e
    ge_ref[...] = jnp.dot(e, Wgv_ref[...], preferred_element_type=jnp.float32)
    gd_ref[...] = jnp.dot(e, Wgd_ref[...], preferred_element_type=jnp.float32)


def _edge_dense(s_src, s_dst, xe, We, Wgv, Wgd):
    E = s_src.shape[0]
    grid = (E // _EBLK,)
    bs64 = pl.BlockSpec((_EBLK, 64), lambda i: (i, 0))
    bw = lambda r, c: pl.BlockSpec((r, c), lambda i: (0, 0))
    e, ge, gd = pl.pallas_call(
        _edge_dense_body,
        grid=grid,
        in_specs=[bs64, bs64, bs64,
                  bw(192, 64), bw(64, 64), bw(64, 64)],
        out_specs=[bs64, bs64, bs64],
        out_shape=[jax.ShapeDtypeStruct((E, 64), jnp.float32)] * 3,
        interpret=_interpret,
    )(s_src, s_dst, xe, We, Wgv, Wgd)
    return e, ge, gd


def _frames(x1, x2, x3):
    eps = 1e-6
    v1 = x3 - x2
    v2 = x1 - x2
    e1 = v1 / (jnp.linalg.norm(v1, axis=-1, keepdims=True) + eps)
    u2 = v2 - e1 * jnp.sum(e1 * v2, axis=-1, keepdims=True)
    e2 = u2 / (jnp.linalg.norm(u2, axis=-1, keepdims=True) + eps)
    e3 = jnp.cross(e1, e2)
    return jnp.stack([e1, e2, e3], axis=-1), x2


def kernel(timestep_encoding, noised_atm_coords, lig_atom_attr, lig_trp_attr,
           lig_af_pair_attr, lig_atom_pair_attr, gather_idx_i_molid,
           gather_idx_ijk_i, gather_idx_ijk_j, gather_idx_ijk_k,
           gather_idx_I_ijk, gather_idx_UI_I, gather_idx_UI_u,
           gather_idx_uv_u, gather_idx_uv_v, num_i, num_I, fourier_B,
           atm_embed_W1, atm_embed_b1, atm_embed_W2, atm_embed_b2,
           trp_embed_W, trp_vembed_W, blk_We, blk_Wgv, blk_Wgd, blk_Wus,
           blk_Wuv, out_atm_W, out_scale_W, out_scale_b):
    N = noised_atm_coords.shape[0]
    M = lig_trp_attr.shape[0]
    ts = timestep_encoding[gather_idx_i_molid]
    ang = 2.0 * np.pi * (ts @ fourier_B)
    tenc = jnp.concatenate([jnp.cos(ang), jnp.sin(ang)], axis=-1)
    h = jnp.concatenate([lig_atom_attr, tenc], axis=-1)
    h = jax.nn.gelu(h @ atm_embed_W1 + atm_embed_b1) @ atm_embed_W2 + atm_embed_b2
    atm_rep = jnp.concatenate([h[:, None, :], jnp.zeros((N, 3, h.shape[-1]), dtype=h.dtype)], axis=1)
    ti = gather_idx_ijk_i[gather_idx_I_ijk]
    tj = gather_idx_ijk_j[gather_idx_I_ijk]
    tk = gather_idx_ijk_k[gather_idx_I_ijk]
    coords = noised_atm_coords
    coords = coords + jnp.asarray(num_i - N + num_I - M).astype(coords.dtype)
    Rt, tt = _frames(coords[ti], coords[tj], coords[tk])
    trp_s = lig_trp_attr @ trp_embed_W
    trp_v = (lig_trp_attr @ trp_vembed_W).reshape(M, 3, -1)
    trp_rep = jnp.concatenate([trp_s[:, None, :], jnp.einsum('nij,njd->nid', Rt, trp_v)], axis=1)
    x = jnp.concatenate([atm_rep, trp_rep], axis=0)
    src = jnp.concatenate([gather_idx_UI_I + N, gather_idx_UI_u, gather_idx_uv_u])
    dst = jnp.concatenate([gather_idx_UI_u, gather_idx_UI_I + N, gather_idx_uv_v])
    x_edge = jnp.concatenate([lig_af_pair_attr, lig_af_pair_attr, lig_atom_pair_attr], axis=0)
    Nt = N + M
    E = src.shape[0]
    G1 = -(-E // (_NW * _KG))
    B1 = _NW * _KG * G1
    src_p = jnp.pad(src, (0, B1 - E))
    dst_p = jnp.pad(dst, (0, B1 - E))
    H = Nt // 2
    dh0 = jnp.where(dst_p < H, dst_p, H)
    dh1 = jnp.where(dst_p >= H, dst_p - H, H)
    idx2 = jnp.stack([dh0, dh1])
    zeros_tbl = jnp.zeros((H + 8, 64), jnp.float32)
    n_stacks = blk_We.shape[0]
    eye = jnp.broadcast_to(jnp.eye(3, dtype=coords.dtype), (N, 3, 3))
    for l in range(n_stacks):
        Rt, tt = _frames(coords[ti], coords[tj], coords[tk])
        R_all = jnp.concatenate([eye, Rt], axis=0)
        t_all = jnp.concatenate([coords, tt], axis=0)
        s = x[:, 0]
        v = x[:, 1:]
        sv = jnp.concatenate([s, v.reshape(Nt, 192)], axis=1)
        sg = _sc_gather(sv, src_p, G1)
        s128 = jnp.concatenate([s, jnp.zeros((Nt, 64), jnp.float32)], axis=1)
        dg = _sc_gather(s128, dst_p, G1)
        e, ge, gdd = _edge_dense(sg[:E, :64], dg[:E, :64], x_edge,
                                 blk_We[l], blk_Wgv[l], blk_Wgd[l])
        agg_s = jax.ops.segment_sum(e, dst, num_segments=Nt)
        rel = t_all[src] - t_all[dst]
        rel_loc = jnp.einsum('eji,ej->ei', R_all[dst], rel)
        vmsg = sg[:E, 64:].reshape(E, 3, 64) * ge[:, None, :] + rel_loc[:, :, None] * gdd[:, None, :]
        agg_v = jax.ops.segment_sum(vmsg, dst, num_segments=Nt)
        s = s + jax.nn.gelu(jnp.concatenate([s, agg_s], axis=-1) @ blk_Wus[l])
        v = v + jnp.einsum('nid,de->nie', agg_v, blk_Wuv[l])
        x = jnp.concatenate([s[:, None, :], v], axis=1)
        atm = x[:N]
        scale = jax.nn.sigmoid(atm[:, 0] @ out_scale_W + out_scale_b) * 10.0
        ddir = (atm[:, 1:] @ out_atm_W)[..., 0]
        coords = coords + ddir * scale
    return coords
